# baseline (speedup 1.0000x reference)
import functools

import jax
import jax.numpy as jnp
from jax import lax
from jax.experimental import pallas as pl
from jax.experimental.pallas import tpu as pltpu
from jax.experimental.pallas import tpu_sc as plsc

WH, WW = 16, 16
N = WH * WW
NN = N * N
NUM_HEADS = 12
TABLE_LEN = (2 * WH - 1) * (2 * WW - 1)
B_WINDOWS = 128

NC, NS = 2, 16
NW = NC * NS
ROWS_PER_W = 16
LANE = 128
PER_W = ROWS_PER_W * LANE


def _sc_gather_bias(table_flat, rel2d):
    mesh = plsc.VectorSubcoreMesh(core_axis_name="c", subcore_axis_name="s")

    @functools.partial(
        pl.kernel,
        out_type=jax.ShapeDtypeStruct((NUM_HEADS, NN // LANE, LANE),
                                      jnp.float32),
        mesh=mesh,
        scratch_types=[
            pltpu.VMEM((ROWS_PER_W, LANE), jnp.int32),
            pltpu.VMEM((NUM_HEADS * ROWS_PER_W, LANE), jnp.int32),
            pltpu.VMEM((NUM_HEADS * ROWS_PER_W, LANE), jnp.float32),
            pltpu.SemaphoreType.DMA,
        ],
    )
    def k(tab_hbm, rel_hbm, bias_hbm, rel_v, idx_v, val_v, sem):
        wid = lax.axis_index("s") * NC + lax.axis_index("c")
        row0 = wid * ROWS_PER_W
        pltpu.sync_copy(rel_hbm.at[pl.ds(row0, ROWS_PER_W)], rel_v)

        def idx_body(t, _):
            h = t // ROWS_PER_W
            r = t % ROWS_PER_W
            for j in range(LANE // 16):
                sl = pl.ds(j * 16, 16)
                idx_v[t, sl] = rel_v[r, sl] + h * TABLE_LEN
            return 0

        lax.fori_loop(0, NUM_HEADS * ROWS_PER_W, idx_body, 0)

        def gather_body(h, _):
            base = h * ROWS_PER_W

            def fire(r, _):
                pltpu.async_copy(tab_hbm.at[idx_v.at[base + r]],
                                 val_v.at[base + r], sem)
                return 0

            lax.fori_loop(0, ROWS_PER_W, fire, 0)
            blk = pl.ds(base, ROWS_PER_W)
            pltpu.make_async_copy(bias_hbm.at[0, pl.ds(0, ROWS_PER_W)],
                                  val_v.at[blk], sem).wait()
            pltpu.sync_copy(val_v.at[blk],
                            bias_hbm.at[h, pl.ds(row0, ROWS_PER_W)])
            return 0

        lax.fori_loop(0, NUM_HEADS, gather_body, 0)

    return k(table_flat, rel2d)


BB = 8


def _tc_add_body(bias_ref, attn_ref, out_ref):
    out_ref[...] = attn_ref[...] + bias_ref[...][None]


def _tc_add(attn, bias):
    return pl.pallas_call(
        _tc_add_body,
        grid=(NUM_HEADS, B_WINDOWS // BB),
        in_specs=[
            pl.BlockSpec((1, N, N), lambda h, b: (h, 0, 0)),
            pl.BlockSpec((BB, 1, N, N), lambda h, b: (b, h, 0, 0)),
        ],
        out_specs=pl.BlockSpec((BB, 1, N, N), lambda h, b: (b, h, 0, 0)),
        out_shape=jax.ShapeDtypeStruct(attn.shape, attn.dtype),
    )(bias, attn)


def kernel(attn, table, rel_index):
    table_flat = table.T.reshape(-1)
    rel2d = rel_index.reshape(NN // LANE, LANE).astype(jnp.int32)
    bias = _sc_gather_bias(table_flat, rel2d)
    return _tc_add(attn, bias.reshape(NUM_HEADS, N, N))


# speedup vs baseline: 1.9818x; 1.9818x over previous
import functools

import jax
import jax.numpy as jnp
from jax import lax
from jax.experimental import pallas as pl
from jax.experimental.pallas import tpu as pltpu
from jax.experimental.pallas import tpu_sc as plsc

WH, WW = 16, 16
N = WH * WW
NN = N * N
NUM_HEADS = 12
TABLE_LEN = (2 * WH - 1) * (2 * WW - 1)
B_WINDOWS = 128

NC, NS = 2, 16
NW = NC * NS
ROWS_PER_W = 16
LANE = 128
PER_W = ROWS_PER_W * LANE


def _sc_gather_bias(table_flat, rel2d):
    mesh = plsc.VectorSubcoreMesh(core_axis_name="c", subcore_axis_name="s")

    @functools.partial(
        pl.kernel,
        out_type=jax.ShapeDtypeStruct((NUM_HEADS, NN // LANE, LANE),
                                      jnp.float32),
        mesh=mesh,
        compiler_params=pltpu.CompilerParams(needs_layout_passes=False),
        scratch_types=[
            pltpu.VMEM((NUM_HEADS * TABLE_LEN,), jnp.float32),
            pltpu.VMEM((ROWS_PER_W, LANE), jnp.int32),
            pltpu.VMEM((NUM_HEADS * ROWS_PER_W, LANE), jnp.float32),
            pltpu.SemaphoreType.DMA,
        ],
    )
    def k(tab_hbm, rel_hbm, bias_hbm, tab_v, rel_v, val_v, sem):
        wid = lax.axis_index("s") * NC + lax.axis_index("c")
        row0 = wid * ROWS_PER_W
        pltpu.sync_copy(tab_hbm, tab_v)
        pltpu.sync_copy(rel_hbm.at[pl.ds(row0, ROWS_PER_W)], rel_v)

        def head_body(h, _):
            hoff = h * TABLE_LEN

            def row_body(r, _):
                for j in range(LANE // 16):
                    sl = pl.ds(j * 16, 16)
                    vals = plsc.load_gather(tab_v, [rel_v[r, sl] + hoff])
                    val_v[h * ROWS_PER_W + r, sl] = vals
                return 0

            lax.fori_loop(0, ROWS_PER_W, row_body, 0)
            pltpu.async_copy(val_v.at[pl.ds(h * ROWS_PER_W, ROWS_PER_W)],
                             bias_hbm.at[h, pl.ds(row0, ROWS_PER_W)], sem)
            return 0

        lax.fori_loop(0, NUM_HEADS, head_body, 0)
        pltpu.make_async_copy(
            bias_hbm.at[0, pl.ds(0, NUM_HEADS * ROWS_PER_W)], val_v,
            sem).wait()

    return k(table_flat, rel2d)


BB = 8


def _tc_add_body(bias_ref, attn_ref, out_ref):
    out_ref[...] = attn_ref[...] + bias_ref[...][None]


def _tc_add(attn, bias):
    return pl.pallas_call(
        _tc_add_body,
        grid=(NUM_HEADS, B_WINDOWS // BB),
        in_specs=[
            pl.BlockSpec((1, N, N), lambda h, b: (h, 0, 0)),
            pl.BlockSpec((BB, 1, N, N), lambda h, b: (b, h, 0, 0)),
        ],
        out_specs=pl.BlockSpec((BB, 1, N, N), lambda h, b: (b, h, 0, 0)),
        out_shape=jax.ShapeDtypeStruct(attn.shape, attn.dtype),
    )(bias, attn)


def kernel(attn, table, rel_index):
    table_flat = table.T.reshape(-1)
    rel2d = rel_index.reshape(NN // LANE, LANE).astype(jnp.int32)
    bias = _sc_gather_bias(table_flat, rel2d)
    return _tc_add(attn, bias.reshape(NUM_HEADS, N, N))


# speedup vs baseline: 2.1722x; 1.0961x over previous
import functools

import jax
import jax.numpy as jnp
from jax import lax
from jax.experimental import pallas as pl
from jax.experimental.pallas import tpu as pltpu
from jax.experimental.pallas import tpu_sc as plsc

WH, WW = 16, 16
N = WH * WW
NN = N * N
NUM_HEADS = 12
TABLE_LEN = (2 * WH - 1) * (2 * WW - 1)
B_WINDOWS = 128

NC, NS = 2, 16
NW = NC * NS
ROWS_PER_W = 16
LANE = 128
PER_W = ROWS_PER_W * LANE


def _sc_gather_bias(table_flat, rel2d):
    mesh = plsc.VectorSubcoreMesh(core_axis_name="c", subcore_axis_name="s")

    @functools.partial(
        pl.kernel,
        out_type=jax.ShapeDtypeStruct((NUM_HEADS, NN // LANE, LANE),
                                      jnp.float32),
        mesh=mesh,
        compiler_params=pltpu.CompilerParams(needs_layout_passes=False),
        scratch_types=[
            pltpu.VMEM((NUM_HEADS * TABLE_LEN,), jnp.float32),
            pltpu.VMEM((ROWS_PER_W, LANE), jnp.int32),
            pltpu.VMEM((NUM_HEADS * ROWS_PER_W, LANE), jnp.float32),
            pltpu.SemaphoreType.DMA,
        ],
    )
    def k(tab_hbm, rel_hbm, bias_hbm, tab_v, rel_v, val_v, sem):
        wid = lax.axis_index("s") * NC + lax.axis_index("c")
        row0 = wid * ROWS_PER_W
        pltpu.sync_copy(tab_hbm, tab_v)
        pltpu.sync_copy(rel_hbm.at[pl.ds(row0, ROWS_PER_W)], rel_v)

        def head_body(h, _):
            hoff = h * TABLE_LEN

            def row_body(r, _):
                for j in range(LANE // 16):
                    sl = pl.ds(j * 16, 16)
                    vals = plsc.load_gather(tab_v, [rel_v[r, sl] + hoff])
                    val_v[h * ROWS_PER_W + r, sl] = vals
                return 0

            lax.fori_loop(0, ROWS_PER_W, row_body, 0)
            pltpu.async_copy(val_v.at[pl.ds(h * ROWS_PER_W, ROWS_PER_W)],
                             bias_hbm.at[h, pl.ds(row0, ROWS_PER_W)], sem)
            return 0

        lax.fori_loop(0, NUM_HEADS, head_body, 0)
        pltpu.make_async_copy(
            bias_hbm.at[0, pl.ds(0, NUM_HEADS * ROWS_PER_W)], val_v,
            sem).wait()

    return k(table_flat, rel2d)


BB = 2


def _tc_add_body(bias_ref, attn_ref, out_ref):
    out_ref[...] = attn_ref[...] + bias_ref[...][None]


def _tc_add(attn, bias):
    return pl.pallas_call(
        _tc_add_body,
        grid=(B_WINDOWS // BB,),
        in_specs=[
            pl.BlockSpec((NUM_HEADS, N, N), lambda b: (0, 0, 0)),
            pl.BlockSpec((BB, NUM_HEADS, N, N), lambda b: (b, 0, 0, 0)),
        ],
        out_specs=pl.BlockSpec((BB, NUM_HEADS, N, N), lambda b: (b, 0, 0, 0)),
        out_shape=jax.ShapeDtypeStruct(attn.shape, attn.dtype),
    )(bias, attn)


def kernel(attn, table, rel_index):
    table_flat = table.T.reshape(-1)
    rel2d = rel_index.reshape(NN // LANE, LANE).astype(jnp.int32)
    bias = _sc_gather_bias(table_flat, rel2d)
    return _tc_add(attn, bias.reshape(NUM_HEADS, N, N))


# speedup vs baseline: 2.1865x; 1.0066x over previous
import functools

import jax
import jax.numpy as jnp
from jax import lax
from jax.experimental import pallas as pl
from jax.experimental.pallas import tpu as pltpu
from jax.experimental.pallas import tpu_sc as plsc

WH, WW = 16, 16
N = WH * WW
NN = N * N
NUM_HEADS = 12
TABLE_LEN = (2 * WH - 1) * (2 * WW - 1)
B_WINDOWS = 128

NC, NS = 2, 16
NW = NC * NS
ROWS_PER_W = 16
LANE = 128
PER_W = ROWS_PER_W * LANE


def _sc_gather_bias(table_flat, rel2d):
    mesh = plsc.VectorSubcoreMesh(core_axis_name="c", subcore_axis_name="s")

    @functools.partial(
        pl.kernel,
        out_type=jax.ShapeDtypeStruct((NUM_HEADS, NN // LANE, LANE),
                                      jnp.float32),
        mesh=mesh,
        compiler_params=pltpu.CompilerParams(needs_layout_passes=False),
        scratch_types=[
            pltpu.VMEM((NUM_HEADS * TABLE_LEN,), jnp.float32),
            pltpu.VMEM((ROWS_PER_W, LANE), jnp.int32),
            pltpu.VMEM((NUM_HEADS * ROWS_PER_W, LANE), jnp.float32),
            pltpu.SemaphoreType.DMA,
        ],
    )
    def k(tab_hbm, rel_hbm, bias_hbm, tab_v, rel_v, val_v, sem):
        wid = lax.axis_index("s") * NC + lax.axis_index("c")
        row0 = wid * ROWS_PER_W
        pltpu.sync_copy(tab_hbm, tab_v)
        pltpu.sync_copy(rel_hbm.at[pl.ds(row0, ROWS_PER_W)], rel_v)

        def head_body(h, _):
            hoff = h * TABLE_LEN

            def row_body(r, _):
                for j in range(LANE // 16):
                    sl = pl.ds(j * 16, 16)
                    vals = plsc.load_gather(tab_v, [rel_v[r, sl] + hoff])
                    val_v[h * ROWS_PER_W + r, sl] = vals
                return 0

            lax.fori_loop(0, ROWS_PER_W, row_body, 0)
            pltpu.async_copy(val_v.at[pl.ds(h * ROWS_PER_W, ROWS_PER_W)],
                             bias_hbm.at[h, pl.ds(row0, ROWS_PER_W)], sem)
            return 0

        lax.fori_loop(0, NUM_HEADS, head_body, 0)
        pltpu.make_async_copy(
            bias_hbm.at[0, pl.ds(0, NUM_HEADS * ROWS_PER_W)], val_v,
            sem).wait()

    return k(table_flat, rel2d)


BB = 4


def _tc_add_body(bias_ref, attn_ref, out_ref):
    out_ref[...] = attn_ref[...] + bias_ref[...][None]


def _tc_add(attn, bias):
    return pl.pallas_call(
        _tc_add_body,
        grid=(B_WINDOWS // BB,),
        in_specs=[
            pl.BlockSpec((NUM_HEADS, N, N), lambda b: (0, 0, 0)),
            pl.BlockSpec((BB, NUM_HEADS, N, N), lambda b: (b, 0, 0, 0)),
        ],
        out_specs=pl.BlockSpec((BB, NUM_HEADS, N, N), lambda b: (b, 0, 0, 0)),
        out_shape=jax.ShapeDtypeStruct(attn.shape, attn.dtype),
    )(bias, attn)


def kernel(attn, table, rel_index):
    table_flat = table.T.reshape(-1)
    rel2d = rel_index.reshape(NN // LANE, LANE).astype(jnp.int32)
    bias = _sc_gather_bias(table_flat, rel2d)
    return _tc_add(attn, bias.reshape(NUM_HEADS, N, N))


# speedup vs baseline: 2.2373x; 1.0232x over previous
import functools

import jax
import jax.numpy as jnp
from jax import lax
from jax.experimental import pallas as pl
from jax.experimental.pallas import tpu as pltpu
from jax.experimental.pallas import tpu_sc as plsc

WH, WW = 16, 16
N = WH * WW
NN = N * N
NUM_HEADS = 12
TABLE_LEN = (2 * WH - 1) * (2 * WW - 1)
B_WINDOWS = 128

NC, NS = 2, 16
NW = NC * NS
ROWS_PER_W = 16
LANE = 128
PER_W = ROWS_PER_W * LANE


def _sc_gather_bias(table_flat, rel2d):
    mesh = plsc.VectorSubcoreMesh(core_axis_name="c", subcore_axis_name="s")

    @functools.partial(
        pl.kernel,
        out_type=jax.ShapeDtypeStruct((NUM_HEADS, NN // LANE, LANE),
                                      jnp.float32),
        mesh=mesh,
        compiler_params=pltpu.CompilerParams(needs_layout_passes=False),
        scratch_types=[
            pltpu.VMEM((NUM_HEADS * TABLE_LEN,), jnp.float32),
            pltpu.VMEM((ROWS_PER_W, LANE), jnp.int32),
            pltpu.VMEM((NUM_HEADS * ROWS_PER_W, LANE), jnp.float32),
            pltpu.SemaphoreType.DMA,
        ],
    )
    def k(tab_hbm, rel_hbm, bias_hbm, tab_v, rel_v, val_v, sem):
        wid = lax.axis_index("s") * NC + lax.axis_index("c")
        row0 = wid * ROWS_PER_W
        pltpu.sync_copy(tab_hbm, tab_v)
        pltpu.sync_copy(rel_hbm.at[pl.ds(row0, ROWS_PER_W)], rel_v)

        def row_body(r, _):
            for j in range(LANE // 16):
                sl = pl.ds(j * 16, 16)
                rel16 = rel_v[r, sl]
                for h in range(NUM_HEADS):
                    val_v[h * ROWS_PER_W + r, sl] = plsc.load_gather(
                        tab_v, [rel16 + h * TABLE_LEN])
            return 0

        lax.fori_loop(0, ROWS_PER_W, row_body, 0)
        for h in range(NUM_HEADS):
            pltpu.async_copy(val_v.at[pl.ds(h * ROWS_PER_W, ROWS_PER_W)],
                             bias_hbm.at[h, pl.ds(row0, ROWS_PER_W)], sem)
        pltpu.make_async_copy(
            bias_hbm.at[0, pl.ds(0, NUM_HEADS * ROWS_PER_W)], val_v,
            sem).wait()

    return k(table_flat, rel2d)


BB = 4


def _tc_add_body(bias_ref, attn_ref, out_ref):
    out_ref[...] = attn_ref[...] + bias_ref[...][None]


def _tc_add(attn, bias):
    return pl.pallas_call(
        _tc_add_body,
        grid=(B_WINDOWS // BB,),
        in_specs=[
            pl.BlockSpec((NUM_HEADS, N, N), lambda b: (0, 0, 0)),
            pl.BlockSpec((BB, NUM_HEADS, N, N), lambda b: (b, 0, 0, 0)),
        ],
        out_specs=pl.BlockSpec((BB, NUM_HEADS, N, N), lambda b: (b, 0, 0, 0)),
        out_shape=jax.ShapeDtypeStruct(attn.shape, attn.dtype),
    )(bias, attn)


def kernel(attn, table, rel_index):
    table_flat = table.T.reshape(-1)
    rel2d = rel_index.reshape(NN // LANE, LANE).astype(jnp.int32)
    bias = _sc_gather_bias(table_flat, rel2d)
    return _tc_add(attn, bias.reshape(NUM_HEADS, N, N))


# speedup vs baseline: 2.2387x; 1.0006x over previous
import functools

import jax
import jax.numpy as jnp
from jax import lax
from jax.experimental import pallas as pl
from jax.experimental.pallas import tpu as pltpu
from jax.experimental.pallas import tpu_sc as plsc

WH, WW = 16, 16
N = WH * WW
NN = N * N
NUM_HEADS = 12
TABLE_LEN = (2 * WH - 1) * (2 * WW - 1)
B_WINDOWS = 128

NC, NS = 2, 16
NW = NC * NS
ROWS_PER_W = 16
LANE = 128
PER_W = ROWS_PER_W * LANE


def _sc_gather_bias(table_flat, rel2d):
    mesh = plsc.VectorSubcoreMesh(core_axis_name="c", subcore_axis_name="s")

    @functools.partial(
        pl.kernel,
        out_type=jax.ShapeDtypeStruct((NUM_HEADS, NN // LANE, LANE),
                                      jnp.float32),
        mesh=mesh,
        compiler_params=pltpu.CompilerParams(needs_layout_passes=False),
        scratch_types=[
            pltpu.VMEM((NUM_HEADS * TABLE_LEN,), jnp.float32),
            pltpu.VMEM((ROWS_PER_W, LANE), jnp.int32),
            pltpu.VMEM((NUM_HEADS * ROWS_PER_W, LANE), jnp.float32),
            pltpu.SemaphoreType.DMA,
        ],
    )
    def k(tab_hbm, rel_hbm, bias_hbm, tab_v, rel_v, val_v, sem):
        wid = lax.axis_index("s") * NC + lax.axis_index("c")
        row0 = wid * ROWS_PER_W
        pltpu.async_copy(tab_hbm, tab_v, sem)
        pltpu.async_copy(rel_hbm.at[pl.ds(row0, ROWS_PER_W)], rel_v,
                         sem).wait()
        pltpu.make_async_copy(tab_hbm, tab_v, sem).wait()

        def row_body(r, _):
            for j in range(LANE // 16):
                sl = pl.ds(j * 16, 16)
                rel16 = rel_v[r, sl] * NUM_HEADS
                for h in range(NUM_HEADS):
                    val_v[h * ROWS_PER_W + r, sl] = plsc.load_gather(
                        tab_v, [rel16 + h])
            return 0

        lax.fori_loop(0, ROWS_PER_W, row_body, 0)
        for h in range(NUM_HEADS):
            pltpu.async_copy(val_v.at[pl.ds(h * ROWS_PER_W, ROWS_PER_W)],
                             bias_hbm.at[h, pl.ds(row0, ROWS_PER_W)], sem)
        pltpu.make_async_copy(
            bias_hbm.at[0, pl.ds(0, NUM_HEADS * ROWS_PER_W)], val_v,
            sem).wait()

    return k(table_flat, rel2d)


BB = 4


def _tc_add_body(bias_ref, attn_ref, out_ref):
    out_ref[...] = attn_ref[...] + bias_ref[...][None]


def _tc_add(attn, bias):
    return pl.pallas_call(
        _tc_add_body,
        grid=(B_WINDOWS // BB,),
        in_specs=[
            pl.BlockSpec((NUM_HEADS, N, N), lambda b: (0, 0, 0)),
            pl.BlockSpec((BB, NUM_HEADS, N, N), lambda b: (b, 0, 0, 0)),
        ],
        out_specs=pl.BlockSpec((BB, NUM_HEADS, N, N), lambda b: (b, 0, 0, 0)),
        out_shape=jax.ShapeDtypeStruct(attn.shape, attn.dtype),
    )(bias, attn)


def kernel(attn, table, rel_index):
    table_flat = table.reshape(-1)
    rel2d = rel_index.reshape(NN // LANE, LANE).astype(jnp.int32)
    bias = _sc_gather_bias(table_flat, rel2d)
    return _tc_add(attn, bias.reshape(NUM_HEADS, N, N))


# speedup vs baseline: 2.2400x; 1.0006x over previous
import functools

import jax
import jax.numpy as jnp
from jax import lax
from jax.experimental import pallas as pl
from jax.experimental.pallas import tpu as pltpu
from jax.experimental.pallas import tpu_sc as plsc

WH, WW = 16, 16
N = WH * WW
NN = N * N
NUM_HEADS = 12
TABLE_LEN = (2 * WH - 1) * (2 * WW - 1)
B_WINDOWS = 128

NC, NS = 2, 16
NW = NC * NS
ROWS_PER_W = 16
LANE = 128
PER_W = ROWS_PER_W * LANE


def _sc_gather_bias(table_flat, rel2d):
    mesh = plsc.VectorSubcoreMesh(core_axis_name="c", subcore_axis_name="s")

    @functools.partial(
        pl.kernel,
        out_type=jax.ShapeDtypeStruct((NUM_HEADS, NN // LANE, LANE),
                                      jnp.float32),
        mesh=mesh,
        compiler_params=pltpu.CompilerParams(needs_layout_passes=False),
        scratch_types=[
            pltpu.VMEM((NUM_HEADS * TABLE_LEN,), jnp.float32),
            pltpu.VMEM((ROWS_PER_W, LANE), jnp.int32),
            pltpu.VMEM((NUM_HEADS * ROWS_PER_W, LANE), jnp.float32),
            pltpu.SemaphoreType.DMA,
        ],
    )
    def k(tab_hbm, rel_hbm, bias_hbm, tab_v, rel_v, val_v, sem):
        wid = lax.axis_index("s") * NC + lax.axis_index("c")
        row0 = wid * ROWS_PER_W
        pltpu.async_copy(tab_hbm, tab_v, sem)
        pltpu.async_copy(rel_hbm.at[pl.ds(row0, ROWS_PER_W)], rel_v,
                         sem).wait()
        pltpu.make_async_copy(tab_hbm, tab_v, sem).wait()

        def pre_body(r, _):
            for j in range(LANE // 16):
                sl = pl.ds(j * 16, 16)
                rel_v[r, sl] = rel_v[r, sl] * NUM_HEADS
            return 0

        lax.fori_loop(0, ROWS_PER_W, pre_body, 0)

        for h in range(NUM_HEADS):
            def row_body(r, _):
                for j in range(LANE // 16):
                    sl = pl.ds(j * 16, 16)
                    val_v[h * ROWS_PER_W + r, sl] = plsc.load_gather(
                        tab_v, [rel_v[r, sl] + h])
                return 0

            lax.fori_loop(0, ROWS_PER_W, row_body, 0)
            pltpu.async_copy(val_v.at[pl.ds(h * ROWS_PER_W, ROWS_PER_W)],
                             bias_hbm.at[h, pl.ds(row0, ROWS_PER_W)], sem)
        pltpu.make_async_copy(
            bias_hbm.at[0, pl.ds(0, NUM_HEADS * ROWS_PER_W)], val_v,
            sem).wait()

    return k(table_flat, rel2d)


BB = 4


def _tc_add_body(bias_ref, attn_ref, out_ref):
    out_ref[...] = attn_ref[...] + bias_ref[...][None]


def _tc_add(attn, bias):
    return pl.pallas_call(
        _tc_add_body,
        grid=(B_WINDOWS // BB,),
        in_specs=[
            pl.BlockSpec((NUM_HEADS, N, N), lambda b: (0, 0, 0)),
            pl.BlockSpec((BB, NUM_HEADS, N, N), lambda b: (b, 0, 0, 0)),
        ],
        out_specs=pl.BlockSpec((BB, NUM_HEADS, N, N), lambda b: (b, 0, 0, 0)),
        out_shape=jax.ShapeDtypeStruct(attn.shape, attn.dtype),
    )(bias, attn)


def kernel(attn, table, rel_index):
    table_flat = table.reshape(-1)
    rel2d = rel_index.reshape(NN // LANE, LANE).astype(jnp.int32)
    bias = _sc_gather_bias(table_flat, rel2d)
    return _tc_add(attn, bias.reshape(NUM_HEADS, N, N))
